# Initial kernel scaffold; baseline (speedup 1.0000x reference)
import jax
import jax.numpy as jnp
from jax.experimental import pallas as pl


def kernel(x, e, edge_index, eU_W, eU_b, eV_W, eV_b, nU_W, nU_b, nV_W, nV_b, ln_e_g, ln_e_b, ln_n_g, ln_n_b):
    raise NotImplementedError("write your pallas kernel here")


# speedup vs baseline: 1.4069x; 1.4069x over previous
import functools

import jax
import jax.numpy as jnp
from jax import lax
from jax.experimental import pallas as pl
from jax.experimental.pallas import tpu as pltpu
from jax.experimental.pallas import tpu_sc as plsc

N = 10000
E = 160000
H = 256

NC = 2
NS = 16
L = 16
NW = NC * NS

CH = 128
NCHUNK = E // CH
CPW_G = -(-NCHUNK // NW)
CPW_S = -(-NCHUNK // NS)
HALF = N // NC
ZR = ((HALF + NS - 1) // NS // 8) * 8 + 8

@functools.cache
def _make_sc_gather():
    mesh = plsc.VectorSubcoreMesh(core_axis_name="c", subcore_axis_name="s")
    return functools.partial(
        pl.kernel,
        out_type=[jax.ShapeDtypeStruct((E, H), jnp.float32)] * 3,
        mesh=mesh,
        scratch_types=[
            pltpu.VMEM((CH,), jnp.int32),
            pltpu.VMEM((CH,), jnp.int32),
            pltpu.VMEM((CH, H), jnp.float32),
            pltpu.VMEM((CH, H), jnp.float32),
            pltpu.VMEM((CH, H), jnp.float32),
            pltpu.SemaphoreType.DMA,
        ],
    )(_sc_gather_body)


def _sc_gather_body(vx_hbm, vx2_hbm, src_hbm, dst_hbm, oS, oD, o2,
                    si_v, di_v, bS, bD, b2, sem):
    wid = lax.axis_index("s") * NC + lax.axis_index("c")

    @pl.loop(0, CPW_G)
    def _(j):
        chunk = wid + j * NW

        @pl.when(chunk < NCHUNK)
        def _():
            base = chunk * CH
            pltpu.sync_copy(src_hbm.at[pl.ds(base, CH)], si_v)
            pltpu.sync_copy(dst_hbm.at[pl.ds(base, CH)], di_v)
            cS = pltpu.async_copy(vx_hbm.at[si_v], bS, sem)
            cD = pltpu.async_copy(vx_hbm.at[di_v], bD, sem)
            c2 = pltpu.async_copy(vx2_hbm.at[si_v], b2, sem)
            cS.wait()
            cD.wait()
            c2.wait()
            pltpu.sync_copy(bS, oS.at[pl.ds(base, CH)])
            pltpu.sync_copy(bD, oD.at[pl.ds(base, CH)])
            pltpu.sync_copy(b2, o2.at[pl.ds(base, CH)])


def _node_mm_body(x_ref, wv_ref, bv_ref, wu_ref, bu_ref, w2_ref, b2_ref,
                  vx_ref, ux_ref, vx2_ref):
    xb = x_ref[...]
    vx_ref[...] = jnp.dot(xb, wv_ref[...],
                          preferred_element_type=jnp.float32) + bv_ref[...]
    ux_ref[...] = jnp.dot(xb, wu_ref[...],
                          preferred_element_type=jnp.float32) + bu_ref[...]
    vx2_ref[...] = jnp.dot(xb, w2_ref[...],
                           preferred_element_type=jnp.float32) + b2_ref[...]


def _edge_body(dst_ref, e_ref, vxs_ref, vxd_ref, vx2s_ref, w_ref, b_ref,
               g_ref, be_ref, enew_ref, agg_ref, gn_scr):
    eb = e_ref[...]
    et = (jnp.dot(eb, w_ref[...], preferred_element_type=jnp.float32)
          + b_ref[...] + vxs_ref[...] + vxd_ref[...])
    gate = jax.nn.sigmoid(et)
    gn_scr[...] = vx2s_ref[...] * gate
    m = jnp.mean(et, axis=-1, keepdims=True)
    var = jnp.mean((et - m) * (et - m), axis=-1, keepdims=True)
    ln = (et - m) * lax.rsqrt(var + 1e-5) * g_ref[...] + be_ref[...]
    enew_ref[...] = eb + jnp.maximum(ln, 0.0)

    @pl.when(pl.program_id(0) == 0)
    def _():
        agg_ref[...] = jnp.zeros_like(agg_ref)

    def _acc(k, _):
        d = dst_ref[0, 0, k]
        agg_ref[pl.ds(d, 1), :] += gn_scr[pl.ds(k, 1), :]
        return 0

    lax.fori_loop(0, BE, _acc, 0)


def _node_fin_body(x_ref, ux_ref, agg_ref, g_ref, b_ref, xnew_ref):
    xt = ux_ref[...] + agg_ref[...]
    m = jnp.mean(xt, axis=-1, keepdims=True)
    var = jnp.mean((xt - m) * (xt - m), axis=-1, keepdims=True)
    ln = (xt - m) * lax.rsqrt(var + 1e-5) * g_ref[...] + b_ref[...]
    xnew_ref[...] = x_ref[...] + jnp.maximum(ln, 0.0)


def _row_spec(bm):
    return pl.BlockSpec((bm, H), lambda i: (i, 0))


def _full_spec(shape):
    return pl.BlockSpec(shape, lambda i: (0,) * len(shape))


BN = 2000
BE = 2000


def kernel(x, e, edge_index, eU_W, eU_b, eV_W, eV_b, nU_W, nU_b, nV_W, nV_b,
           ln_e_g, ln_e_b, ln_n_g, ln_n_b):
    f32 = jnp.float32
    src = edge_index[0]
    dst = edge_index[1]

    node_mm = pl.pallas_call(
        _node_mm_body,
        grid=(N // BN,),
        in_specs=[_row_spec(BN)] + [_full_spec((H, H)), _full_spec((1, H))] * 3,
        out_specs=[_row_spec(BN)] * 3,
        out_shape=[jax.ShapeDtypeStruct((N, H), f32)] * 3,
    )
    vx, ux, vx2 = node_mm(
        x,
        eV_W.T, eV_b.reshape(1, H),
        nU_W.T, nU_b.reshape(1, H),
        nV_W.T, nV_b.reshape(1, H),
    )

    vxs, vxd, vx2s = _make_sc_gather()(vx, vx2, src, dst)

    edge_tc = pl.pallas_call(
        _edge_body,
        grid=(E // BE,),
        in_specs=[pl.BlockSpec((1, 1, BE), lambda i: (i, 0, 0),
                               memory_space=pltpu.SMEM)]
        + [_row_spec(BE)] * 4
        + [_full_spec((H, H))] + [_full_spec((1, H))] * 3,
        out_specs=[_row_spec(BE), pl.BlockSpec((N, H), lambda i: (0, 0))],
        out_shape=[jax.ShapeDtypeStruct((E, H), f32),
                   jax.ShapeDtypeStruct((N, H), f32)],
        scratch_shapes=[pltpu.VMEM((BE, H), f32)],
    )
    e_new, agg = edge_tc(
        dst.reshape(E // BE, 1, BE), e, vxs, vxd, vx2s,
        eU_W.T, eU_b.reshape(1, H),
        ln_e_g.reshape(1, H), ln_e_b.reshape(1, H),
    )

    node_fin = pl.pallas_call(
        _node_fin_body,
        grid=(N // BN,),
        in_specs=[_row_spec(BN)] * 3 + [_full_spec((1, H))] * 2,
        out_specs=_row_spec(BN),
        out_shape=jax.ShapeDtypeStruct((N, H), f32),
    )
    x_new = node_fin(x, ux, agg, ln_n_g.reshape(1, H), ln_n_b.reshape(1, H))

    return (x_new, e_new)


# speedup vs baseline: 2.1814x; 1.5506x over previous
import functools

import jax
import jax.numpy as jnp
from jax import lax
from jax.experimental import pallas as pl
from jax.experimental.pallas import tpu as pltpu
from jax.experimental.pallas import tpu_sc as plsc

N = 10000
E = 160000
H = 256

NC = 2
NS = 16
L = 16
NW = NC * NS

CH = 128
NCHUNK = E // CH
CPW_G = -(-NCHUNK // NW)
CPW_S = -(-NCHUNK // NS)
HALF = N // NC
ZR = ((HALF + NS - 1) // NS // 8) * 8 + 8

@functools.cache
def _make_sc_gather():
    mesh = plsc.VectorSubcoreMesh(core_axis_name="c", subcore_axis_name="s")
    return functools.partial(
        pl.kernel,
        out_type=[jax.ShapeDtypeStruct((E, H), jnp.float32)] * 3,
        mesh=mesh,
        scratch_types=[
            pltpu.VMEM((CH,), jnp.int32),
            pltpu.VMEM((CH,), jnp.int32),
            pltpu.VMEM((CH, H), jnp.float32),
            pltpu.VMEM((CH, H), jnp.float32),
            pltpu.VMEM((CH, H), jnp.float32),
            pltpu.SemaphoreType.DMA,
        ],
    )(_sc_gather_body)


def _sc_gather_body(vx_hbm, vx2_hbm, src_hbm, dst_hbm, oS, oD, o2,
                    si_v, di_v, bS, bD, b2, sem):
    wid = lax.axis_index("s") * NC + lax.axis_index("c")

    @pl.loop(0, CPW_G)
    def _(j):
        chunk = wid + j * NW

        @pl.when(chunk < NCHUNK)
        def _():
            base = chunk * CH
            pltpu.sync_copy(src_hbm.at[pl.ds(base, CH)], si_v)
            pltpu.sync_copy(dst_hbm.at[pl.ds(base, CH)], di_v)
            cS = pltpu.async_copy(vx_hbm.at[si_v], bS, sem)
            cD = pltpu.async_copy(vx_hbm.at[di_v], bD, sem)
            c2 = pltpu.async_copy(vx2_hbm.at[si_v], b2, sem)
            cS.wait()
            cD.wait()
            c2.wait()
            pltpu.sync_copy(bS, oS.at[pl.ds(base, CH)])
            pltpu.sync_copy(bD, oD.at[pl.ds(base, CH)])
            pltpu.sync_copy(b2, o2.at[pl.ds(base, CH)])


def _node_mm_body(x_ref, wv_ref, bv_ref, wu_ref, bu_ref, w2_ref, b2_ref,
                  vx_ref, ux_ref, vx2_ref):
    xb = x_ref[...]
    vx_ref[...] = jnp.dot(xb, wv_ref[...],
                          preferred_element_type=jnp.float32) + bv_ref[...]
    ux_ref[...] = jnp.dot(xb, wu_ref[...],
                          preferred_element_type=jnp.float32) + bu_ref[...]
    vx2_ref[...] = jnp.dot(xb, w2_ref[...],
                           preferred_element_type=jnp.float32) + b2_ref[...]


def _edge_body(dst_ref, e_ref, vxs_ref, vxd_ref, vx2s_ref, w_ref, b_ref,
               g_ref, be_ref, enew_ref, agg_ref, agg2_ref, gn_scr):
    eb = e_ref[...]
    et = (jnp.dot(eb, w_ref[...], preferred_element_type=jnp.float32)
          + b_ref[...] + vxs_ref[...] + vxd_ref[...])
    gate = jax.nn.sigmoid(et)
    gn_scr[...] = vx2s_ref[...] * gate
    m = jnp.mean(et, axis=-1, keepdims=True)
    var = jnp.mean((et - m) * (et - m), axis=-1, keepdims=True)
    ln = (et - m) * lax.rsqrt(var + 1e-5) * g_ref[...] + be_ref[...]
    enew_ref[...] = eb + jnp.maximum(ln, 0.0)

    @pl.when(pl.program_id(0) == 0)
    def _():
        agg_ref[...] = jnp.zeros_like(agg_ref)
        agg2_ref[...] = jnp.zeros_like(agg2_ref)

    def _acc(k2, _):
        k = k2 * 2
        d0 = dst_ref[0, 0, k]
        d1 = dst_ref[0, 0, k + 1]
        agg_ref[pl.ds(d0, 1), :] += gn_scr[pl.ds(k, 1), :]
        agg2_ref[pl.ds(d1, 1), :] += gn_scr[pl.ds(k + 1, 1), :]
        return 0

    lax.fori_loop(0, BE // 2, _acc, 0, unroll=4)


def _node_fin_body(x_ref, ux_ref, agg_ref, agg2_ref, g_ref, b_ref, xnew_ref):
    xt = ux_ref[...] + agg_ref[...] + agg2_ref[...]
    m = jnp.mean(xt, axis=-1, keepdims=True)
    var = jnp.mean((xt - m) * (xt - m), axis=-1, keepdims=True)
    ln = (xt - m) * lax.rsqrt(var + 1e-5) * g_ref[...] + b_ref[...]
    xnew_ref[...] = x_ref[...] + jnp.maximum(ln, 0.0)


def _row_spec(bm):
    return pl.BlockSpec((bm, H), lambda i: (i, 0))


def _full_spec(shape):
    return pl.BlockSpec(shape, lambda i: (0,) * len(shape))


BN = 2000
BE = 2000


def kernel(x, e, edge_index, eU_W, eU_b, eV_W, eV_b, nU_W, nU_b, nV_W, nV_b,
           ln_e_g, ln_e_b, ln_n_g, ln_n_b):
    f32 = jnp.float32
    src = edge_index[0]
    dst = edge_index[1]

    node_mm = pl.pallas_call(
        _node_mm_body,
        grid=(N // BN,),
        in_specs=[_row_spec(BN)] + [_full_spec((H, H)), _full_spec((1, H))] * 3,
        out_specs=[_row_spec(BN)] * 3,
        out_shape=[jax.ShapeDtypeStruct((N, H), f32)] * 3,
    )
    vx, ux, vx2 = node_mm(
        x,
        eV_W.T, eV_b.reshape(1, H),
        nU_W.T, nU_b.reshape(1, H),
        nV_W.T, nV_b.reshape(1, H),
    )

    vxs, vxd, vx2s = _make_sc_gather()(vx, vx2, src, dst)

    edge_tc = pl.pallas_call(
        _edge_body,
        grid=(E // BE,),
        in_specs=[pl.BlockSpec((1, 1, BE), lambda i: (i, 0, 0),
                               memory_space=pltpu.SMEM)]
        + [_row_spec(BE)] * 4
        + [_full_spec((H, H))] + [_full_spec((1, H))] * 3,
        out_specs=[_row_spec(BE),
                   pl.BlockSpec((N, H), lambda i: (0, 0)),
                   pl.BlockSpec((N, H), lambda i: (0, 0))],
        out_shape=[jax.ShapeDtypeStruct((E, H), f32),
                   jax.ShapeDtypeStruct((N, H), f32),
                   jax.ShapeDtypeStruct((N, H), f32)],
        scratch_shapes=[pltpu.VMEM((BE, H), f32)],
    )
    e_new, agg, agg2 = edge_tc(
        dst.reshape(E // BE, 1, BE), e, vxs, vxd, vx2s,
        eU_W.T, eU_b.reshape(1, H),
        ln_e_g.reshape(1, H), ln_e_b.reshape(1, H),
    )

    node_fin = pl.pallas_call(
        _node_fin_body,
        grid=(N // BN,),
        in_specs=[_row_spec(BN)] * 4 + [_full_spec((1, H))] * 2,
        out_specs=_row_spec(BN),
        out_shape=jax.ShapeDtypeStruct((N, H), f32),
    )
    x_new = node_fin(x, ux, agg, agg2,
                     ln_n_g.reshape(1, H), ln_n_b.reshape(1, H))

    return (x_new, e_new)
